# Initial kernel scaffold; baseline (speedup 1.0000x reference)
import jax
import jax.numpy as jnp
from jax.experimental import pallas as pl


def kernel(x, edge_attr, edge_index, shapes, e_W1, e_b1, e_W2, e_b2, e_g, e_bt, n_W1, n_b1, n_W2, n_b2, n_g, n_bt):
    raise NotImplementedError("write your pallas kernel here")


# speedup vs baseline: 1.8828x; 1.8828x over previous
import functools

import jax
import jax.numpy as jnp
from jax import lax
from jax.experimental import pallas as pl
from jax.experimental.pallas import tpu as pltpu
from jax.experimental.pallas import tpu_sc as plsc

_NUM_SC_CORES = 2
_NUM_SC_SUBCORES = 16
_GATHER_W = 80


def _proj_body(x_ref, wa_ref, wb_ref, xa_ref, xb_ref):
    x = x_ref[...]
    xa_ref[...] = jnp.dot(x, wa_ref[...], preferred_element_type=jnp.float32)
    xb_ref[...] = jnp.dot(x, wb_ref[...], preferred_element_type=jnp.float32)


def _proj(x, wa, wb, block=2000):
    n, c = x.shape
    return pl.pallas_call(
        _proj_body,
        grid=(n // block,),
        in_specs=[
            pl.BlockSpec((block, c), lambda i: (i, 0)),
            pl.BlockSpec((c, c), lambda i: (0, 0)),
            pl.BlockSpec((c, c), lambda i: (0, 0)),
        ],
        out_specs=[
            pl.BlockSpec((block, c), lambda i: (i, 0)),
            pl.BlockSpec((block, c), lambda i: (i, 0)),
        ],
        out_shape=[jax.ShapeDtypeStruct((n, c), jnp.float32)] * 2,
    )(x, wa, wb)


def _sc_gather(xa, xb, dst, src):
    e = dst.shape[0]
    c = xa.shape[1]
    w = 128
    tiles = _NUM_SC_CORES * _NUM_SC_SUBCORES
    e_pad = ((e + w * tiles - 1) // (w * tiles)) * (w * tiles)
    mesh = plsc.VectorSubcoreMesh(core_axis_name="c", subcore_axis_name="s")
    dst2 = jnp.pad(dst, (0, e_pad - e)).reshape(1, e_pad)
    src2 = jnp.pad(src, (0, e_pad - e)).reshape(1, e_pad)

    @functools.partial(
        pl.kernel,
        out_type=[jax.ShapeDtypeStruct((e_pad, c), jnp.float32)] * 2,
        mesh=mesh,
    )
    def k(xa_hbm, xb_hbm, dst_hbm, src_hbm, ga_hbm, gb_hbm):
        def body(d_v, s_v, ga_v, gb_v):
            pltpu.sync_copy(xa_hbm.at[d_v.at[0]], ga_v)
            pltpu.sync_copy(xb_hbm.at[s_v.at[0]], gb_v)

        pltpu.emit_pipeline(
            body,
            grid=(e_pad // w,),
            in_specs=[
                pl.BlockSpec((1, w), lambda i: (0, i)),
                pl.BlockSpec((1, w), lambda i: (0, i)),
            ],
            out_specs=[
                pl.BlockSpec((w, c), lambda i: (i, 0)),
                pl.BlockSpec((w, c), lambda i: (i, 0)),
            ],
            core_axis_name=("c", "s"),
            dimension_semantics=(pltpu.PARALLEL,),
        )(dst_hbm, src_hbm, ga_hbm, gb_hbm)

    ga, gb = k(xa, xb, dst2, src2)
    return ga[:e], gb[:e]


def _edge_body(ga_ref, gb_ref, ea_ref, w1_ref, b1_ref, w2_ref, b2_ref,
               g_ref, bt_ref, out_ref):
    ea = ea_ref[...]
    pre = (ga_ref[...] + gb_ref[...]
           + jnp.dot(ea, w1_ref[...], preferred_element_type=jnp.float32)
           + b1_ref[...])
    h = pre * jax.nn.sigmoid(pre)
    h2 = jnp.dot(h, w2_ref[...], preferred_element_type=jnp.float32) + b2_ref[...]
    mu = jnp.mean(h2, axis=-1, keepdims=True)
    zc = h2 - mu
    var = jnp.mean(zc * zc, axis=-1, keepdims=True)
    out_ref[...] = zc * lax.rsqrt(var + 1e-5) * g_ref[...] + bt_ref[...] + ea


def _edge_mlp(ga, gb, ea, w1c, b1, w2, b2, g, bt, block=2000):
    e, c = ea.shape
    row = lambda v: v.reshape(1, c)
    bspec = pl.BlockSpec((block, c), lambda i: (i, 0))
    wspec = pl.BlockSpec((c, c), lambda i: (0, 0))
    vspec = pl.BlockSpec((1, c), lambda i: (0, 0))
    return pl.pallas_call(
        _edge_body,
        grid=(e // block,),
        in_specs=[bspec, bspec, bspec, wspec, vspec, wspec, vspec, vspec, vspec],
        out_specs=bspec,
        out_shape=jax.ShapeDtypeStruct((e, c), jnp.float32),
    )(ga, gb, ea, w1c, row(b1), w2, row(b2), row(g), row(bt))


def _sc_scatter(edges, dst, zeros):
    e, c = edges.shape
    n = zeros.shape[0]
    w = _GATHER_W
    nw = _NUM_SC_CORES * _NUM_SC_SUBCORES
    per_tile = e // nw
    chunks = per_tile // w
    rows_per_sub = n // _NUM_SC_SUBCORES
    assert rows_per_sub % 8 == 0 and n % _NUM_SC_SUBCORES == 0
    mesh = plsc.VectorSubcoreMesh(core_axis_name="c", subcore_axis_name="s")

    @functools.partial(
        pl.kernel,
        out_type=jax.ShapeDtypeStruct((_NUM_SC_CORES, n, c), jnp.float32),
        mesh=mesh,
        scratch_types=[
            pltpu.VMEM((w,), jnp.int32),
            pltpu.VMEM((w, c), jnp.float32),
            pltpu.VMEM_SHARED((n, c), jnp.float32),
        ],
    )
    def k(edges_hbm, dst_hbm, zeros_hbm, out_hbm, idx_v, rows_v, acc_sh):
        cid = lax.axis_index("c")
        sid = lax.axis_index("s")
        wid = sid * _NUM_SC_CORES + cid
        row0 = sid * rows_per_sub
        pltpu.sync_copy(zeros_hbm.at[pl.ds(row0, rows_per_sub)],
                        acc_sh.at[pl.ds(row0, rows_per_sub)])
        plsc.subcore_barrier()

        @pl.loop(0, chunks)
        def _(ci):
            base = wid * per_tile + ci * w
            pltpu.sync_copy(dst_hbm.at[pl.ds(base, w)], idx_v)
            pltpu.sync_copy(edges_hbm.at[pl.ds(base, w)], rows_v)
            pltpu.sync_copy(rows_v, acc_sh.at[idx_v], add=True)

        plsc.subcore_barrier()
        pltpu.sync_copy(acc_sh.at[pl.ds(row0, rows_per_sub)],
                        out_hbm.at[cid].at[pl.ds(row0, rows_per_sub)])

    return k(edges, dst, zeros)


def _node_body(x_ref, a0_ref, a1_ref, wa_ref, wb_ref, b1_ref, w2_ref, b2_ref,
               g_ref, bt_ref, out_ref):
    x = x_ref[...]
    agg = a0_ref[...] + a1_ref[...]
    pre = (jnp.dot(x, wa_ref[...], preferred_element_type=jnp.float32)
           + jnp.dot(agg, wb_ref[...], preferred_element_type=jnp.float32)
           + b1_ref[...])
    h = pre * jax.nn.sigmoid(pre)
    h2 = jnp.dot(h, w2_ref[...], preferred_element_type=jnp.float32) + b2_ref[...]
    mu = jnp.mean(h2, axis=-1, keepdims=True)
    zc = h2 - mu
    var = jnp.mean(zc * zc, axis=-1, keepdims=True)
    out_ref[...] = zc * lax.rsqrt(var + 1e-5) * g_ref[...] + bt_ref[...] + x


def _node_mlp(x, a0, a1, wa, wb, b1, w2, b2, g, bt, block=2000):
    n, c = x.shape
    row = lambda v: v.reshape(1, c)
    bspec = pl.BlockSpec((block, c), lambda i: (i, 0))
    wspec = pl.BlockSpec((c, c), lambda i: (0, 0))
    vspec = pl.BlockSpec((1, c), lambda i: (0, 0))
    return pl.pallas_call(
        _node_body,
        grid=(n // block,),
        in_specs=[bspec, bspec, bspec, wspec, wspec, vspec, wspec, vspec,
                  vspec, vspec],
        out_specs=bspec,
        out_shape=jax.ShapeDtypeStruct((n, c), jnp.float32),
    )(x, a0, a1, wa, wb, row(b1), w2, row(b2), row(g), row(bt))


def kernel(x, edge_attr, edge_index, shapes, e_W1, e_b1, e_W2, e_b2, e_g,
           e_bt, n_W1, n_b1, n_W2, n_b2, n_g, n_bt):
    n, c = x.shape
    num_layers = e_W1.shape[0]
    src = edge_index[0]
    dst = edge_index[1]
    n_pad = ((n + 127) // 128) * 128
    zeros = jnp.zeros((n_pad, c), jnp.float32)
    x_out = x
    ea = edge_attr
    for l in range(num_layers):
        w1 = e_W1[l]
        xa, xb = _proj(x_out, w1[:c], w1[c:2 * c])
        ga, gb = _sc_gather(xa, xb, dst, src)
        ea = _edge_mlp(ga, gb, ea, w1[2 * c:], e_b1[l], e_W2[l], e_b2[l],
                       e_g[l], e_bt[l])
        agg2 = _sc_scatter(ea, dst, zeros)
        nw1 = n_W1[l]
        x_out = _node_mlp(x_out, agg2[0, :n], agg2[1, :n], nw1[:c], nw1[c:],
                          n_b1[l], n_W2[l], n_b2[l], n_g[l], n_bt[l])
    return (x_out, ea)


# speedup vs baseline: 2.4794x; 1.3169x over previous
import functools

import jax
import jax.numpy as jnp
from jax import lax
from jax.experimental import pallas as pl
from jax.experimental.pallas import tpu as pltpu
from jax.experimental.pallas import tpu_sc as plsc

_NUM_SC_CORES = 2
_NUM_SC_SUBCORES = 16
_GATHER_W = 80


def _proj_body(x_ref, w_ref, out_ref):
    out_ref[0] = jnp.dot(x_ref[...], w_ref[0],
                         preferred_element_type=jnp.float32)


def _proj(x_pad, wstack, block=2048):
    n_pad, c = x_pad.shape
    return pl.pallas_call(
        _proj_body,
        grid=(2, n_pad // block),
        in_specs=[
            pl.BlockSpec((block, c), lambda i, j: (j, 0)),
            pl.BlockSpec((1, c, c), lambda i, j: (i, 0, 0)),
        ],
        out_specs=pl.BlockSpec((1, block, c), lambda i, j: (i, j, 0)),
        out_shape=jax.ShapeDtypeStruct((2, n_pad, c), jnp.float32),
    )(x_pad, wstack)


def _sc_gather(tables, idx2, e):
    _, n_pad, c = tables.shape
    e_pad = idx2.shape[1]
    w = 128
    ns = _NUM_SC_SUBCORES
    per_sub = e_pad // ns
    chunks = per_sub // w
    rows_tab = n_pad // ns
    mesh = plsc.VectorSubcoreMesh(core_axis_name="c", subcore_axis_name="s")

    @functools.partial(
        pl.kernel,
        out_type=jax.ShapeDtypeStruct((_NUM_SC_CORES, e_pad, c), jnp.float32),
        mesh=mesh,
        scratch_types=[
            pltpu.VMEM((w,), jnp.int32),
            pltpu.VMEM((w, c), jnp.float32),
            pltpu.VMEM_SHARED((n_pad, c), jnp.float32),
        ],
    )
    def k(tab_hbm, idx_hbm, out_hbm, idx_v, rows_v, tab_sh):
        cid = lax.axis_index("c")
        sid = lax.axis_index("s")
        pltpu.sync_copy(tab_hbm.at[cid].at[pl.ds(sid * rows_tab, rows_tab)],
                        tab_sh.at[pl.ds(sid * rows_tab, rows_tab)])
        plsc.subcore_barrier()

        @pl.loop(0, chunks)
        def _(ci):
            base = sid * per_sub + ci * w
            pltpu.sync_copy(idx_hbm.at[cid].at[pl.ds(base, w)], idx_v)
            pltpu.sync_copy(tab_sh.at[idx_v], rows_v)
            pltpu.sync_copy(rows_v, out_hbm.at[cid].at[pl.ds(base, w)])

    out = k(tables, idx2)
    return out[0, :e], out[1, :e]


def _edge_body(ga_ref, gb_ref, ea_ref, w1_ref, b1_ref, w2_ref, b2_ref,
               g_ref, bt_ref, out_ref):
    ea = ea_ref[...]
    pre = (ga_ref[...] + gb_ref[...]
           + jnp.dot(ea, w1_ref[...], preferred_element_type=jnp.float32)
           + b1_ref[...])
    h = pre * jax.nn.sigmoid(pre)
    h2 = jnp.dot(h, w2_ref[...], preferred_element_type=jnp.float32) + b2_ref[...]
    mu = jnp.mean(h2, axis=-1, keepdims=True)
    zc = h2 - mu
    var = jnp.mean(zc * zc, axis=-1, keepdims=True)
    out_ref[...] = zc * lax.rsqrt(var + 1e-5) * g_ref[...] + bt_ref[...] + ea


def _edge_mlp(ga, gb, ea, w1c, b1, w2, b2, g, bt, block=2000):
    e, c = ea.shape
    row = lambda v: v.reshape(1, c)
    bspec = pl.BlockSpec((block, c), lambda i: (i, 0))
    wspec = pl.BlockSpec((c, c), lambda i: (0, 0))
    vspec = pl.BlockSpec((1, c), lambda i: (0, 0))
    return pl.pallas_call(
        _edge_body,
        grid=(e // block,),
        in_specs=[bspec, bspec, bspec, wspec, vspec, wspec, vspec, vspec, vspec],
        out_specs=bspec,
        out_shape=jax.ShapeDtypeStruct((e, c), jnp.float32),
    )(ga, gb, ea, w1c, row(b1), w2, row(b2), row(g), row(bt))


def _sc_scatter(edges, dst, zeros):
    e, c = edges.shape
    n = zeros.shape[0]
    w = _GATHER_W
    nw = _NUM_SC_CORES * _NUM_SC_SUBCORES
    per_tile = e // nw
    chunks = per_tile // w
    rows_per_sub = n // _NUM_SC_SUBCORES
    assert rows_per_sub % 8 == 0 and n % _NUM_SC_SUBCORES == 0
    mesh = plsc.VectorSubcoreMesh(core_axis_name="c", subcore_axis_name="s")

    @functools.partial(
        pl.kernel,
        out_type=jax.ShapeDtypeStruct((_NUM_SC_CORES, n, c), jnp.float32),
        mesh=mesh,
        scratch_types=[
            pltpu.VMEM((w,), jnp.int32),
            pltpu.VMEM((w, c), jnp.float32),
            pltpu.VMEM_SHARED((n, c), jnp.float32),
        ],
    )
    def k(edges_hbm, dst_hbm, zeros_hbm, out_hbm, idx_v, rows_v, acc_sh):
        cid = lax.axis_index("c")
        sid = lax.axis_index("s")
        wid = sid * _NUM_SC_CORES + cid
        row0 = sid * rows_per_sub
        pltpu.sync_copy(zeros_hbm.at[pl.ds(row0, rows_per_sub)],
                        acc_sh.at[pl.ds(row0, rows_per_sub)])
        plsc.subcore_barrier()

        @pl.loop(0, chunks)
        def _(ci):
            base = wid * per_tile + ci * w
            pltpu.sync_copy(dst_hbm.at[pl.ds(base, w)], idx_v)
            pltpu.sync_copy(edges_hbm.at[pl.ds(base, w)], rows_v)
            pltpu.sync_copy(rows_v, acc_sh.at[idx_v], add=True)

        plsc.subcore_barrier()
        pltpu.sync_copy(acc_sh.at[pl.ds(row0, rows_per_sub)],
                        out_hbm.at[cid].at[pl.ds(row0, rows_per_sub)])

    return k(edges, dst, zeros)


def _node_body(x_ref, a0_ref, a1_ref, wa_ref, wb_ref, b1_ref, w2_ref, b2_ref,
               g_ref, bt_ref, out_ref):
    x = x_ref[...]
    agg = a0_ref[...] + a1_ref[...]
    pre = (jnp.dot(x, wa_ref[...], preferred_element_type=jnp.float32)
           + jnp.dot(agg, wb_ref[...], preferred_element_type=jnp.float32)
           + b1_ref[...])
    h = pre * jax.nn.sigmoid(pre)
    h2 = jnp.dot(h, w2_ref[...], preferred_element_type=jnp.float32) + b2_ref[...]
    mu = jnp.mean(h2, axis=-1, keepdims=True)
    zc = h2 - mu
    var = jnp.mean(zc * zc, axis=-1, keepdims=True)
    out_ref[...] = zc * lax.rsqrt(var + 1e-5) * g_ref[...] + bt_ref[...] + x


def _node_mlp(x, a0, a1, wa, wb, b1, w2, b2, g, bt, block=2000):
    n, c = x.shape
    row = lambda v: v.reshape(1, c)
    bspec = pl.BlockSpec((block, c), lambda i: (i, 0))
    wspec = pl.BlockSpec((c, c), lambda i: (0, 0))
    vspec = pl.BlockSpec((1, c), lambda i: (0, 0))
    return pl.pallas_call(
        _node_body,
        grid=(n // block,),
        in_specs=[bspec, bspec, bspec, wspec, wspec, vspec, wspec, vspec,
                  vspec, vspec],
        out_specs=bspec,
        out_shape=jax.ShapeDtypeStruct((n, c), jnp.float32),
    )(x, a0, a1, wa, wb, row(b1), w2, row(b2), row(g), row(bt))


def kernel(x, edge_attr, edge_index, shapes, e_W1, e_b1, e_W2, e_b2, e_g,
           e_bt, n_W1, n_b1, n_W2, n_b2, n_g, n_bt):
    n, c = x.shape
    num_layers = e_W1.shape[0]
    src = edge_index[0]
    dst = edge_index[1]
    e = src.shape[0]
    n_pad = ((n + 2047) // 2048) * 2048
    e_pad = ((e + 2047) // 2048) * 2048
    idx2 = jnp.stack([jnp.pad(dst, (0, e_pad - e)),
                      jnp.pad(src, (0, e_pad - e))])
    zeros = jnp.zeros((n_pad, c), jnp.float32)
    x_out = x
    ea = edge_attr
    for l in range(num_layers):
        w1 = e_W1[l]
        x_pad = jnp.pad(x_out, ((0, n_pad - n), (0, 0)))
        tables = _proj(x_pad, jnp.stack([w1[:c], w1[c:2 * c]]))
        ga, gb = _sc_gather(tables, idx2, e)
        ea = _edge_mlp(ga, gb, ea, w1[2 * c:], e_b1[l], e_W2[l], e_b2[l],
                       e_g[l], e_bt[l])
        agg2 = _sc_scatter(ea, dst, zeros)
        nw1 = n_W1[l]
        x_out = _node_mlp(x_out, agg2[0, :n], agg2[1, :n], nw1[:c], nw1[c:],
                          n_b1[l], n_W2[l], n_b2[l], n_g[l], n_bt[l])
    return (x_out, ea)


# speedup vs baseline: 3.0728x; 1.2394x over previous
import functools

import jax
import jax.numpy as jnp
from jax import lax
from jax.experimental import pallas as pl
from jax.experimental.pallas import tpu as pltpu
from jax.experimental.pallas import tpu_sc as plsc

_NUM_SC_CORES = 2
_NUM_SC_SUBCORES = 16
_GATHER_W = 80


def _proj_body(x_ref, w_ref, out_ref):
    out_ref[0] = jnp.dot(x_ref[...], w_ref[0],
                         preferred_element_type=jnp.float32)


def _proj(x_pad, wstack, block=2048):
    n_pad, c = x_pad.shape
    return pl.pallas_call(
        _proj_body,
        grid=(2, n_pad // block),
        in_specs=[
            pl.BlockSpec((block, c), lambda i, j: (j, 0)),
            pl.BlockSpec((1, c, c), lambda i, j: (i, 0, 0)),
        ],
        out_specs=pl.BlockSpec((1, block, c), lambda i, j: (i, j, 0)),
        out_shape=jax.ShapeDtypeStruct((2, n_pad, c), jnp.float32),
    )(x_pad, wstack)


def _sc_gather(tables, idx2):
    _, n_pad, c = tables.shape
    e = idx2.shape[1]
    w = 128
    ns = _NUM_SC_SUBCORES
    total_chunks = e // w
    per_sub = (total_chunks + ns - 1) // ns
    rows_tab = n_pad // ns
    mesh = plsc.VectorSubcoreMesh(core_axis_name="c", subcore_axis_name="s")

    @functools.partial(
        pl.kernel,
        out_type=jax.ShapeDtypeStruct((_NUM_SC_CORES, e, c), jnp.float32),
        mesh=mesh,
        scratch_types=[
            pltpu.VMEM((w,), jnp.int32),
            pltpu.VMEM((w, c), jnp.float32),
            pltpu.VMEM_SHARED((n_pad, c), jnp.float32),
        ],
    )
    def k(tab_hbm, idx_hbm, out_hbm, idx_v, rows_v, tab_sh):
        cid = lax.axis_index("c")
        sid = lax.axis_index("s")
        pltpu.sync_copy(tab_hbm.at[cid].at[pl.ds(sid * rows_tab, rows_tab)],
                        tab_sh.at[pl.ds(sid * rows_tab, rows_tab)])
        plsc.subcore_barrier()

        @pl.loop(0, per_sub)
        def _(ci):
            chunk = sid + ci * ns

            @pl.when(chunk < total_chunks)
            def _():
                base = chunk * w
                pltpu.sync_copy(idx_hbm.at[cid].at[pl.ds(base, w)], idx_v)
                pltpu.sync_copy(tab_sh.at[idx_v], rows_v)
                pltpu.sync_copy(rows_v, out_hbm.at[cid].at[pl.ds(base, w)])

    return k(tables, idx2)


def _edge_body(ga_ref, gb_ref, ea_ref, w1_ref, b1_ref, w2_ref, b2_ref,
               g_ref, bt_ref, out_ref):
    ea = ea_ref[...]
    pre = (ga_ref[0] + gb_ref[0]
           + jnp.dot(ea, w1_ref[...], preferred_element_type=jnp.float32)
           + b1_ref[...])
    h = pre * jax.nn.sigmoid(pre)
    h2 = jnp.dot(h, w2_ref[...], preferred_element_type=jnp.float32) + b2_ref[...]
    mu = jnp.mean(h2, axis=-1, keepdims=True)
    zc = h2 - mu
    var = jnp.mean(zc * zc, axis=-1, keepdims=True)
    out_ref[...] = zc * lax.rsqrt(var + 1e-5) * g_ref[...] + bt_ref[...] + ea


def _edge_mlp(gab, ea, w1c, b1, w2, b2, g, bt, block=2000):
    e, c = ea.shape
    row = lambda v: v.reshape(1, c)
    bspec = pl.BlockSpec((block, c), lambda i: (i, 0))
    aspec = pl.BlockSpec((1, block, c), lambda i: (0, i, 0))
    bspec2 = pl.BlockSpec((1, block, c), lambda i: (1, i, 0))
    wspec = pl.BlockSpec((c, c), lambda i: (0, 0))
    vspec = pl.BlockSpec((1, c), lambda i: (0, 0))
    return pl.pallas_call(
        _edge_body,
        grid=(e // block,),
        in_specs=[aspec, bspec2, bspec, wspec, vspec, wspec, vspec, vspec,
                  vspec],
        out_specs=bspec,
        out_shape=jax.ShapeDtypeStruct((e, c), jnp.float32),
    )(gab, gab, ea, w1c, row(b1), w2, row(b2), row(g), row(bt))


def _sc_scatter(edges, dst, zeros):
    e, c = edges.shape
    n = zeros.shape[0]
    w = _GATHER_W
    nw = _NUM_SC_CORES * _NUM_SC_SUBCORES
    per_tile = e // nw
    chunks = per_tile // w
    rows_per_sub = n // _NUM_SC_SUBCORES
    assert rows_per_sub % 8 == 0 and n % _NUM_SC_SUBCORES == 0
    mesh = plsc.VectorSubcoreMesh(core_axis_name="c", subcore_axis_name="s")

    @functools.partial(
        pl.kernel,
        out_type=jax.ShapeDtypeStruct((_NUM_SC_CORES, n, c), jnp.float32),
        mesh=mesh,
        scratch_types=[
            pltpu.VMEM((w,), jnp.int32),
            pltpu.VMEM((w, c), jnp.float32),
            pltpu.VMEM_SHARED((n, c), jnp.float32),
        ],
    )
    def k(edges_hbm, dst_hbm, zeros_hbm, out_hbm, idx_v, rows_v, acc_sh):
        cid = lax.axis_index("c")
        sid = lax.axis_index("s")
        wid = sid * _NUM_SC_CORES + cid
        row0 = sid * rows_per_sub
        pltpu.sync_copy(zeros_hbm.at[pl.ds(row0, rows_per_sub)],
                        acc_sh.at[pl.ds(row0, rows_per_sub)])
        plsc.subcore_barrier()

        @pl.loop(0, chunks)
        def _(ci):
            base = wid * per_tile + ci * w
            pltpu.sync_copy(dst_hbm.at[pl.ds(base, w)], idx_v)
            pltpu.sync_copy(edges_hbm.at[pl.ds(base, w)], rows_v)
            pltpu.sync_copy(rows_v, acc_sh.at[idx_v], add=True)

        plsc.subcore_barrier()
        pltpu.sync_copy(acc_sh.at[pl.ds(row0, rows_per_sub)],
                        out_hbm.at[cid].at[pl.ds(row0, rows_per_sub)])

    return k(edges, dst, zeros)


def _node_body(x_ref, a0_ref, a1_ref, wa_ref, wb_ref, b1_ref, w2_ref, b2_ref,
               g_ref, bt_ref, out_ref):
    x = x_ref[...]
    agg = a0_ref[0] + a1_ref[0]
    pre = (jnp.dot(x, wa_ref[...], preferred_element_type=jnp.float32)
           + jnp.dot(agg, wb_ref[...], preferred_element_type=jnp.float32)
           + b1_ref[...])
    h = pre * jax.nn.sigmoid(pre)
    h2 = jnp.dot(h, w2_ref[...], preferred_element_type=jnp.float32) + b2_ref[...]
    mu = jnp.mean(h2, axis=-1, keepdims=True)
    zc = h2 - mu
    var = jnp.mean(zc * zc, axis=-1, keepdims=True)
    out_ref[...] = zc * lax.rsqrt(var + 1e-5) * g_ref[...] + bt_ref[...] + x


def _node_mlp(x, agg2, wa, wb, b1, w2, b2, g, bt, block=2000):
    n, c = x.shape
    row = lambda v: v.reshape(1, c)
    bspec = pl.BlockSpec((block, c), lambda i: (i, 0))
    aspec = pl.BlockSpec((1, block, c), lambda i: (0, i, 0))
    aspec2 = pl.BlockSpec((1, block, c), lambda i: (1, i, 0))
    wspec = pl.BlockSpec((c, c), lambda i: (0, 0))
    vspec = pl.BlockSpec((1, c), lambda i: (0, 0))
    return pl.pallas_call(
        _node_body,
        grid=(n // block,),
        in_specs=[bspec, aspec, aspec2, wspec, wspec, vspec, wspec, vspec,
                  vspec, vspec],
        out_specs=bspec,
        out_shape=jax.ShapeDtypeStruct((n, c), jnp.float32),
    )(x, agg2, agg2, wa, wb, row(b1), w2, row(b2), row(g), row(bt))


def kernel(x, edge_attr, edge_index, shapes, e_W1, e_b1, e_W2, e_b2, e_g,
           e_bt, n_W1, n_b1, n_W2, n_b2, n_g, n_bt):
    n, c = x.shape
    num_layers = e_W1.shape[0]
    src = edge_index[0]
    dst = edge_index[1]
    e = src.shape[0]
    assert e % 128 == 0
    n_pad = ((n + 2047) // 2048) * 2048
    idx2 = jnp.stack([dst, src])
    zeros = jnp.zeros((n_pad, c), jnp.float32)
    x_out = x
    ea = edge_attr
    for l in range(num_layers):
        w1 = e_W1[l]
        x_pad = jnp.pad(x_out, ((0, n_pad - n), (0, 0)))
        tables = _proj(x_pad, jnp.stack([w1[:c], w1[c:2 * c]]))
        gab = _sc_gather(tables, idx2)
        ea = _edge_mlp(gab, ea, w1[2 * c:], e_b1[l], e_W2[l], e_b2[l],
                       e_g[l], e_bt[l])
        agg2 = _sc_scatter(ea, dst, zeros)
        nw1 = n_W1[l]
        x_out = _node_mlp(x_out, agg2, nw1[:c], nw1[c:],
                          n_b1[l], n_W2[l], n_b2[l], n_g[l], n_bt[l])
    return (x_out, ea)


# speedup vs baseline: 4.9216x; 1.6016x over previous
import functools

import jax
import jax.numpy as jnp
from jax import lax
from jax.experimental import pallas as pl
from jax.experimental.pallas import tpu as pltpu
from jax.experimental.pallas import tpu_sc as plsc

_NUM_SC_CORES = 2
_NUM_SC_SUBCORES = 16
_SCATTER_W = 40


def _proj_body(x_ref, w_ref, out_ref):
    out_ref[0] = jnp.dot(x_ref[...], w_ref[0],
                         preferred_element_type=jnp.float32)


def _proj(x_pad, wstack, block=2048):
    n_pad, c = x_pad.shape
    return pl.pallas_call(
        _proj_body,
        grid=(2, n_pad // block),
        in_specs=[
            pl.BlockSpec((block, c), lambda i, j: (j, 0)),
            pl.BlockSpec((1, c, c), lambda i, j: (i, 0, 0)),
        ],
        out_specs=pl.BlockSpec((1, block, c), lambda i, j: (i, j, 0)),
        out_shape=jax.ShapeDtypeStruct((2, n_pad, c), jnp.float32),
    )(x_pad, wstack)


_G_NBUF = 2


def _sc_gather(tables, idx2p, e):
    _, n_pad, c = tables.shape
    e_idx = idx2p.shape[1]
    w = 128
    ns = _NUM_SC_SUBCORES
    nb = _G_NBUF
    per_sub = e_idx // (ns * w)
    iters = per_sub // nb
    rows_tab = n_pad // ns
    mesh = plsc.VectorSubcoreMesh(core_axis_name="c", subcore_axis_name="s")

    scratch = ([pltpu.VMEM((w,), jnp.int32)] * nb
               + [pltpu.VMEM((w, c), jnp.float32)] * nb
               + [pltpu.SemaphoreType.DMA] * (2 * nb)
               + [pltpu.VMEM_SHARED((n_pad, c), jnp.float32)])

    @functools.partial(
        pl.kernel,
        out_type=jax.ShapeDtypeStruct((_NUM_SC_CORES, e, c), jnp.float32),
        mesh=mesh,
        scratch_types=scratch,
    )
    def k(tab_hbm, idx_hbm, out_hbm, *sc):
        idx_v = sc[0:nb]
        rows_v = sc[nb:2 * nb]
        isem = sc[2 * nb:3 * nb]
        osem = sc[3 * nb:4 * nb]
        tab_sh = sc[4 * nb]
        cid = lax.axis_index("c")
        sid = lax.axis_index("s")
        pltpu.sync_copy(tab_hbm.at[cid].at[pl.ds(sid * rows_tab, rows_tab)],
                        tab_sh.at[pl.ds(sid * rows_tab, rows_tab)])
        plsc.subcore_barrier()
        start = sid * per_sub

        for u in range(nb):
            pltpu.async_copy(
                idx_hbm.at[cid].at[pl.ds((start + u) * w, w)], idx_v[u],
                isem[u])

        @pl.loop(0, iters)
        def _(ci):
            for u in range(nb):
                chunk = start + ci * nb + u
                base = chunk * w

                @pl.when(jnp.logical_and(ci > 0, base - nb * w < e))
                def _():
                    pltpu.make_async_copy(
                        rows_v[u],
                        out_hbm.at[cid].at[pl.ds(base - nb * w, w)],
                        osem[u]).wait()

                pltpu.make_async_copy(
                    idx_hbm.at[cid].at[pl.ds(base, w)], idx_v[u],
                    isem[u]).wait()
                pltpu.sync_copy(tab_sh.at[idx_v[u]], rows_v[u])

                @pl.when(base < e)
                def _():
                    pltpu.async_copy(
                        rows_v[u], out_hbm.at[cid].at[pl.ds(base, w)],
                        osem[u])

                @pl.when(ci < iters - 1)
                def _():
                    pltpu.async_copy(
                        idx_hbm.at[cid].at[pl.ds(base + nb * w, w)], idx_v[u],
                        isem[u])

        for u in range(nb):
            last = (start + (iters - 1) * nb + u) * w

            @pl.when(last < e)
            def _():
                pltpu.make_async_copy(
                    rows_v[u], out_hbm.at[cid].at[pl.ds(last, w)],
                    osem[u]).wait()

    return k(tables, idx2p)


def _edge_body(ga_ref, gb_ref, ea_ref, w1_ref, b1_ref, w2_ref, b2_ref,
               g_ref, bt_ref, out_ref):
    ea = ea_ref[...]
    pre = (ga_ref[0] + gb_ref[0]
           + jnp.dot(ea, w1_ref[...], preferred_element_type=jnp.float32)
           + b1_ref[...])
    h = pre * jax.nn.sigmoid(pre)
    h2 = jnp.dot(h, w2_ref[...], preferred_element_type=jnp.float32) + b2_ref[...]
    mu = jnp.mean(h2, axis=-1, keepdims=True)
    zc = h2 - mu
    var = jnp.mean(zc * zc, axis=-1, keepdims=True)
    out_ref[...] = zc * lax.rsqrt(var + 1e-5) * g_ref[...] + bt_ref[...] + ea


def _edge_mlp(gab, ea, w1c, b1, w2, b2, g, bt, block=2000):
    e, c = ea.shape
    row = lambda v: v.reshape(1, c)
    bspec = pl.BlockSpec((block, c), lambda i: (i, 0))
    aspec = pl.BlockSpec((1, block, c), lambda i: (0, i, 0))
    bspec2 = pl.BlockSpec((1, block, c), lambda i: (1, i, 0))
    wspec = pl.BlockSpec((c, c), lambda i: (0, 0))
    vspec = pl.BlockSpec((1, c), lambda i: (0, 0))
    return pl.pallas_call(
        _edge_body,
        grid=(e // block,),
        in_specs=[aspec, bspec2, bspec, wspec, vspec, wspec, vspec, vspec,
                  vspec],
        out_specs=bspec,
        out_shape=jax.ShapeDtypeStruct((e, c), jnp.float32),
    )(gab, gab, ea, w1c, row(b1), w2, row(b2), row(g), row(bt))


_S_NBUF = 5


def _sc_scatter(edges, dst, zeros):
    e, c = edges.shape
    n = zeros.shape[0]
    w = _SCATTER_W
    nb = _S_NBUF
    nw = _NUM_SC_CORES * _NUM_SC_SUBCORES
    per_tile = e // nw
    chunks = per_tile // w
    iters = chunks // nb
    assert chunks % nb == 0
    rows_per_sub = n // _NUM_SC_SUBCORES
    assert rows_per_sub % 8 == 0 and n % _NUM_SC_SUBCORES == 0
    mesh = plsc.VectorSubcoreMesh(core_axis_name="c", subcore_axis_name="s")

    scratch = ([pltpu.VMEM((w,), jnp.int32)] * nb
               + [pltpu.VMEM((w, c), jnp.float32)] * nb
               + [pltpu.SemaphoreType.DMA] * (2 * nb)
               + [pltpu.VMEM_SHARED((n, c), jnp.float32)])

    @functools.partial(
        pl.kernel,
        out_type=jax.ShapeDtypeStruct((_NUM_SC_CORES, n, c), jnp.float32),
        mesh=mesh,
        scratch_types=scratch,
    )
    def k(edges_hbm, dst_hbm, zeros_hbm, out_hbm, *sc):
        idx_v = sc[0:nb]
        rows_v = sc[nb:2 * nb]
        isem = sc[2 * nb:3 * nb]
        esem = sc[3 * nb:4 * nb]
        acc_sh = sc[4 * nb]
        cid = lax.axis_index("c")
        sid = lax.axis_index("s")
        wid = sid * _NUM_SC_CORES + cid
        row0 = sid * rows_per_sub
        pltpu.sync_copy(zeros_hbm.at[pl.ds(row0, rows_per_sub)],
                        acc_sh.at[pl.ds(row0, rows_per_sub)])
        plsc.subcore_barrier()
        tbase = wid * per_tile

        for u in range(nb):
            pltpu.async_copy(dst_hbm.at[pl.ds(tbase + u * w, w)], idx_v[u],
                             isem[u])
            pltpu.async_copy(edges_hbm.at[pl.ds(tbase + u * w, w)], rows_v[u],
                             esem[u])

        @pl.loop(0, iters)
        def _(ci):
            for u in range(nb):
                base = tbase + (ci * nb + u) * w
                pltpu.make_async_copy(dst_hbm.at[pl.ds(base, w)], idx_v[u],
                                      isem[u]).wait()
                pltpu.make_async_copy(edges_hbm.at[pl.ds(base, w)], rows_v[u],
                                      esem[u]).wait()
                pltpu.sync_copy(rows_v[u], acc_sh.at[idx_v[u]], add=True)

                @pl.when(ci < iters - 1)
                def _():
                    pltpu.async_copy(
                        dst_hbm.at[pl.ds(base + nb * w, w)], idx_v[u],
                        isem[u])
                    pltpu.async_copy(
                        edges_hbm.at[pl.ds(base + nb * w, w)], rows_v[u],
                        esem[u])

        plsc.subcore_barrier()
        pltpu.sync_copy(acc_sh.at[pl.ds(row0, rows_per_sub)],
                        out_hbm.at[cid].at[pl.ds(row0, rows_per_sub)])

    return k(edges, dst, zeros)


def _node_body(x_ref, a0_ref, a1_ref, wa_ref, wb_ref, b1_ref, w2_ref, b2_ref,
               g_ref, bt_ref, out_ref):
    x = x_ref[...]
    agg = a0_ref[0] + a1_ref[0]
    pre = (jnp.dot(x, wa_ref[...], preferred_element_type=jnp.float32)
           + jnp.dot(agg, wb_ref[...], preferred_element_type=jnp.float32)
           + b1_ref[...])
    h = pre * jax.nn.sigmoid(pre)
    h2 = jnp.dot(h, w2_ref[...], preferred_element_type=jnp.float32) + b2_ref[...]
    mu = jnp.mean(h2, axis=-1, keepdims=True)
    zc = h2 - mu
    var = jnp.mean(zc * zc, axis=-1, keepdims=True)
    out_ref[...] = zc * lax.rsqrt(var + 1e-5) * g_ref[...] + bt_ref[...] + x


def _node_mlp(x, agg2, wa, wb, b1, w2, b2, g, bt, block=2000):
    n, c = x.shape
    row = lambda v: v.reshape(1, c)
    bspec = pl.BlockSpec((block, c), lambda i: (i, 0))
    aspec = pl.BlockSpec((1, block, c), lambda i: (0, i, 0))
    aspec2 = pl.BlockSpec((1, block, c), lambda i: (1, i, 0))
    wspec = pl.BlockSpec((c, c), lambda i: (0, 0))
    vspec = pl.BlockSpec((1, c), lambda i: (0, 0))
    return pl.pallas_call(
        _node_body,
        grid=(n // block,),
        in_specs=[bspec, aspec, aspec2, wspec, wspec, vspec, wspec, vspec,
                  vspec, vspec],
        out_specs=bspec,
        out_shape=jax.ShapeDtypeStruct((n, c), jnp.float32),
    )(x, agg2, agg2, wa, wb, row(b1), w2, row(b2), row(g), row(bt))


def kernel(x, edge_attr, edge_index, shapes, e_W1, e_b1, e_W2, e_b2, e_g,
           e_bt, n_W1, n_b1, n_W2, n_b2, n_g, n_bt):
    n, c = x.shape
    num_layers = e_W1.shape[0]
    src = edge_index[0]
    dst = edge_index[1]
    e = src.shape[0]
    assert e % 128 == 0
    n_pad = ((n + 2047) // 2048) * 2048
    stride = 128 * _NUM_SC_SUBCORES * _G_NBUF
    e_idx = ((e + stride - 1) // stride) * stride
    idx2 = jnp.stack([jnp.pad(dst, (0, e_idx - e)),
                      jnp.pad(src, (0, e_idx - e))])
    zeros = jnp.zeros((n_pad, c), jnp.float32)
    x_out = x
    ea = edge_attr
    for l in range(num_layers):
        w1 = e_W1[l]
        x_pad = jnp.pad(x_out, ((0, n_pad - n), (0, 0)))
        tables = _proj(x_pad, jnp.stack([w1[:c], w1[c:2 * c]]))
        gab = _sc_gather(tables, idx2, e)
        ea = _edge_mlp(gab, ea, w1[2 * c:], e_b1[l], e_W2[l], e_b2[l],
                       e_g[l], e_bt[l])
        agg2 = _sc_scatter(ea, dst, zeros)
        nw1 = n_W1[l]
        x_out = _node_mlp(x_out, agg2, nw1[:c], nw1[c:],
                          n_b1[l], n_W2[l], n_b2[l], n_g[l], n_bt[l])
    return (x_out, ea)
